# Initial kernel scaffold; baseline (speedup 1.0000x reference)
import functools

import jax
import jax.numpy as jnp
from jax import lax
from jax.experimental import pallas as pl
from jax.experimental.pallas import tpu as pltpu
from jax.experimental.pallas import tpu_sc as plsc

_B = 2
_NS = 26
_ND = 13
_V = 100000
_D = 32
_L = 16

_SLOTS = 2 * _L
_NIDX = _B * _SLOTS


def _body(x_hbm, tbl_hbm, dt_hbm, out_hbm, x_v, dt_v, dout_v, idx_v, rows_v, sem):
    cid = lax.axis_index("c")
    sid = lax.axis_index("s")

    @pl.when(jnp.logical_and(cid == 0, sid == 0))
    def _():
        pltpu.sync_copy(x_hbm, x_v)
        pltpu.sync_copy(dt_hbm, dt_v)

        for b in range(_B):
            for c in range(2):
                ids = x_v[b, pl.ds(c * _L, _L)].astype(jnp.int32)
                feat = lax.iota(jnp.int32, (_L,)) + c * _L
                flat = feat * _V + ids
                if c == 1:
                    flat = jnp.where(feat < _NS, flat, 0)
                idx_v[pl.ds(b * _SLOTS + c * _L, _L)] = flat

        pltpu.async_copy(tbl_hbm.at[idx_v], rows_v, sem).wait()

        for b in range(_B):
            for j in range(_ND):
                s = x_v[b, _NS + j]
                for h in range(2):
                    dout_v[b, j, pl.ds(h * _L, _L)] = s * dt_v[j, pl.ds(h * _L, _L)]

        for b in range(_B):
            pltpu.sync_copy(rows_v.at[pl.ds(b * _SLOTS, _NS)],
                            out_hbm.at[b, pl.ds(0, _NS)])
            pltpu.sync_copy(dout_v.at[b], out_hbm.at[b, pl.ds(_NS, _ND)])


_sc_call = functools.partial(
    pl.kernel,
    mesh=plsc.VectorSubcoreMesh(core_axis_name="c", subcore_axis_name="s"),
    out_type=jax.ShapeDtypeStruct((_B, _NS + _ND, _D), jnp.float32),
    scratch_types=[
        pltpu.VMEM((_B, _NS + _ND), jnp.float32),
        pltpu.VMEM((_ND, _D), jnp.float32),
        pltpu.VMEM((_B, _ND, _D), jnp.float32),
        pltpu.VMEM((_NIDX,), jnp.int32),
        pltpu.VMEM((_NIDX, _D), jnp.float32),
        pltpu.SemaphoreType.DMA,
    ],
)(_body)


def kernel(X, sparse_tables, dense_tables):
    tbl = sparse_tables.reshape(_NS * _V, _D)
    dt = dense_tables.reshape(_ND, _D)
    return _sc_call(X, tbl, dt)


# speedup vs baseline: 1.2081x; 1.2081x over previous
import functools

import jax
import jax.numpy as jnp
from jax import lax
from jax.experimental import pallas as pl
from jax.experimental.pallas import tpu as pltpu
from jax.experimental.pallas import tpu_sc as plsc

_B = 2
_NS = 26
_ND = 13
_V = 100000
_D = 32
_L = 16

_SLOTS = 2 * _L
_NIDX = _B * _SLOTS


def _body(x_hbm, tbl_hbm, dt_hbm, out_hbm, x_v, dt_v, comb_v, idx_v, sem):
    cid = lax.axis_index("c")
    sid = lax.axis_index("s")

    @pl.when(jnp.logical_and(cid == 0, sid == 0))
    def _():
        pltpu.sync_copy(x_hbm, x_v)
        pltpu.sync_copy(dt_hbm, dt_v)

        for b in range(_B):
            for c in range(2):
                ids = x_v[b, pl.ds(c * _L, _L)].astype(jnp.int32)
                feat = lax.iota(jnp.int32, _L) + c * _L
                flat = feat * _V + ids
                if c == 1:
                    flat = jnp.where(feat < _NS, flat, 0)
                idx_v[b, pl.ds(c * _L, _L)] = flat

        cps = [
            pltpu.async_copy(tbl_hbm.at[idx_v.at[b]],
                             comb_v.at[b, pl.ds(0, _SLOTS)], sem)
            for b in range(_B)
        ]
        for cp in cps:
            cp.wait()

        for b in range(_B):
            dv = x_v[b, pl.ds(_NS + _ND - _L, _L)]
            for j in range(_ND):
                s = dv[j + _L - _ND]
                for h in range(2):
                    comb_v[b, _NS + j, pl.ds(h * _L, _L)] = (
                        s * dt_v[j, pl.ds(h * _L, _L)])

        pltpu.sync_copy(comb_v, out_hbm)


_sc_call = functools.partial(
    pl.kernel,
    mesh=plsc.VectorSubcoreMesh(core_axis_name="c", subcore_axis_name="s"),
    out_type=jax.ShapeDtypeStruct((_B, _NS + _ND, _D), jnp.float32),
    compiler_params=pltpu.CompilerParams(use_tc_tiling_on_sc=False),
    scratch_types=[
        pltpu.VMEM((_B, _NS + _ND), jnp.float32),
        pltpu.VMEM((_ND, _D), jnp.float32),
        pltpu.VMEM((_B, _NS + _ND, _D), jnp.float32),
        pltpu.VMEM((_B, _SLOTS), jnp.int32),
        pltpu.SemaphoreType.DMA,
    ],
)(_body)


def kernel(X, sparse_tables, dense_tables):
    tbl = sparse_tables.reshape(_NS * _V, _D)
    dt = dense_tables.reshape(_ND, _D)
    return _sc_call(X, tbl, dt)


# speedup vs baseline: 4.4282x; 3.6653x over previous
import functools

import jax
import jax.numpy as jnp
from jax import lax
from jax.experimental import pallas as pl
from jax.experimental.pallas import tpu as pltpu
from jax.experimental.pallas import tpu_sc as plsc

_B = 2
_NS = 26
_ND = 13
_NF = _NS + _ND
_V = 100000
_D = 32
_L = 16
_NSLOT = _B * _NS


def _body(x_hbm, tbl_hbm, dt_hbm, out_hbm, x_v, dt_v, comb_v, stage_v, sem):
    cid = lax.axis_index("c")
    sid = lax.axis_index("s")

    @pl.when(jnp.logical_and(cid == 0, sid == 0))
    def _():
        pltpu.sync_copy(x_hbm, x_v)
        pltpu.sync_copy(dt_hbm, dt_v)

        lanes = lax.iota(jnp.int32, _L)

        flats = []
        for b in range(_B):
            for c in range(2):
                ids = x_v[pl.ds(b * _NF + c * _L, _L)].astype(jnp.int32)
                feat = lanes + c * _L
                flats.append((b, c, feat * _V + ids))

        copies = []
        for b, c, flat in flats:
            blk = lax.shift_right_logical(flat, 3)
            for j in range(_L):
                f = c * _L + j
                if f >= _NS:
                    break
                s = b * _NS + f
                off = pl.multiple_of(blk[j] * 8, 8)
                copies.append(pltpu.async_copy(
                    tbl_hbm.at[pl.ds(off, 8)],
                    stage_v.at[pl.ds(s * 8, 8)], sem))

        for b in range(_B):
            dv = x_v[pl.ds(b * _NF + _NF - _L, _L)]
            for j in range(_ND):
                sc = dv[j + _L - _ND]
                for h in range(2):
                    comb_v[pl.ds((b * _NF + _NS + j) * _D + h * _L, _L)] = (
                        sc * dt_v[pl.ds(j * _D + h * _L, _L)])

        for cp in copies:
            cp.wait()

        for b, c, flat in flats:
            sub = jnp.bitwise_and(flat, 7)
            for j in range(_L):
                f = c * _L + j
                if f >= _NS:
                    break
                s = b * _NS + f
                i0 = jnp.broadcast_to(sub[j], (_L,)) + s * 8
                for h in range(2):
                    row = plsc.load_gather(stage_v, [i0, lanes + h * _L])
                    comb_v[pl.ds((b * _NF + f) * _D + h * _L, _L)] = row

        pltpu.sync_copy(comb_v, out_hbm)


_sc_call = functools.partial(
    pl.kernel,
    mesh=plsc.VectorSubcoreMesh(core_axis_name="c", subcore_axis_name="s"),
    out_type=jax.ShapeDtypeStruct((_B * _NF * _D,), jnp.float32),
    compiler_params=pltpu.CompilerParams(needs_layout_passes=False),
    scratch_types=[
        pltpu.VMEM((_B * _NF,), jnp.float32),
        pltpu.VMEM((_ND * _D,), jnp.float32),
        pltpu.VMEM((_B * _NF * _D,), jnp.float32),
        pltpu.VMEM((_NSLOT * 8, _D), jnp.float32),
        pltpu.SemaphoreType.DMA,
    ],
)(_body)


def kernel(X, sparse_tables, dense_tables):
    tbl = sparse_tables.reshape(_NS * _V, _D)
    dt = dense_tables.reshape(_ND * _D)
    out = _sc_call(X.reshape(_B * _NF), tbl, dt)
    return out.reshape(_B, _NF, _D)
